# baseline (speedup 1.0000x reference)
import functools

import jax
import jax.numpy as jnp
from jax.experimental import pallas as pl
from jax.experimental.pallas import tpu as pltpu


def _fused(x_ref, w_ref, emb_ref, b_ref, onehot_ref, out_ref, acc_ref, *,
           s_len, n_chunks):
    k = pl.program_id(0)

    @pl.when(k == 0)
    def _init():
        acc_ref[...] = jnp.zeros_like(acc_ref)

    state_chunk = jnp.sum(x_ref[...], axis=1) * (1.0 / s_len)
    acc_ref[...] += jax.lax.dot_general(
        state_chunk, w_ref[...],
        dimension_numbers=(((1,), (1,)), ((), ())),
        preferred_element_type=jnp.float32)

    @pl.when(k == n_chunks - 1)
    def _finish():
        proj = acc_ref[...] + b_ref[...]
        logits = jax.lax.dot_general(
            proj, emb_ref[...],
            dimension_numbers=(((1,), (1,)), ((), ())),
            preferred_element_type=jnp.float32)
        m = jnp.max(logits, axis=1, keepdims=True)
        lse = jnp.log(jnp.sum(jnp.exp(logits - m), axis=1, keepdims=True)) + m
        logp = logits - lse
        picked = jnp.sum(logp * onehot_ref[...], axis=1, keepdims=True)
        out_ref[...] = jnp.sum(picked, axis=0, keepdims=True) * (-0.001 / logits.shape[0])


def kernel(hidden_states, representations, W, b, current_indices,
           current_expert_idx, current_depth):
    B, S, H = hidden_states.shape
    E = representations.shape[0]
    C = 512
    n = H // C

    emb = jnp.take(representations, current_indices, axis=0)
    onehot = (jax.lax.iota(jnp.int32, E)[None, :]
              == jnp.asarray(current_expert_idx, jnp.int32)).astype(jnp.float32)
    b2 = b.reshape(1, H)

    out = pl.pallas_call(
        functools.partial(_fused, s_len=S, n_chunks=n),
        grid=(n,),
        in_specs=[
            pl.BlockSpec((B, S, C), lambda k: (0, 0, k)),
            pl.BlockSpec((H, C), lambda k: (0, k)),
            pl.BlockSpec((E, H), lambda k: (0, 0)),
            pl.BlockSpec((1, H), lambda k: (0, 0)),
            pl.BlockSpec((1, E), lambda k: (0, 0)),
        ],
        out_specs=pl.BlockSpec((1, 1), lambda k: (0, 0)),
        out_shape=jax.ShapeDtypeStruct((1, 1), jnp.float32),
        scratch_shapes=[pltpu.VMEM((B, H), jnp.float32)],
    )(hidden_states, W, emb, b2, onehot)
    return out[0, 0]


# speedup vs baseline: 1.0406x; 1.0406x over previous
import functools

import jax
import jax.numpy as jnp
from jax.experimental import pallas as pl
from jax.experimental.pallas import tpu as pltpu


def _fused(x_ref, w_ref, emb_ref, b_ref, onehot_ref, out_ref, acc_ref, *,
           s_len, n_chunks):
    k = pl.program_id(0)

    @pl.when(k == 0)
    def _init():
        acc_ref[...] = jnp.zeros_like(acc_ref)

    state_chunk = jnp.sum(x_ref[...], axis=1) * (1.0 / s_len)
    acc_ref[...] += jax.lax.dot_general(
        state_chunk, w_ref[...],
        dimension_numbers=(((1,), (1,)), ((), ())),
        preferred_element_type=jnp.float32)

    @pl.when(k == n_chunks - 1)
    def _finish():
        proj = acc_ref[...] + b_ref[...]
        logits = jax.lax.dot_general(
            proj, emb_ref[...],
            dimension_numbers=(((1,), (1,)), ((), ())),
            preferred_element_type=jnp.float32)
        m = jnp.max(logits, axis=1, keepdims=True)
        lse = jnp.log(jnp.sum(jnp.exp(logits - m), axis=1, keepdims=True)) + m
        logp = logits - lse
        picked = jnp.sum(logp * onehot_ref[...], axis=1, keepdims=True)
        out_ref[...] = jnp.sum(picked, axis=0, keepdims=True) * (-0.001 / logits.shape[0])


def kernel(hidden_states, representations, W, b, current_indices,
           current_expert_idx, current_depth):
    B, S, H = hidden_states.shape
    E = representations.shape[0]
    C = 256
    n = H // C

    emb = jnp.take(representations, current_indices, axis=0)
    onehot = (jax.lax.iota(jnp.int32, E)[None, :]
              == jnp.asarray(current_expert_idx, jnp.int32)).astype(jnp.float32)
    b2 = b.reshape(1, H)

    out = pl.pallas_call(
        functools.partial(_fused, s_len=S, n_chunks=n),
        grid=(n,),
        in_specs=[
            pl.BlockSpec((B, S, C), lambda k: (0, 0, k)),
            pl.BlockSpec((H, C), lambda k: (0, k)),
            pl.BlockSpec((E, H), lambda k: (0, 0)),
            pl.BlockSpec((1, H), lambda k: (0, 0)),
            pl.BlockSpec((1, E), lambda k: (0, 0)),
        ],
        out_specs=pl.BlockSpec((1, 1), lambda k: (0, 0)),
        out_shape=jax.ShapeDtypeStruct((1, 1), jnp.float32),
        scratch_shapes=[pltpu.VMEM((B, H), jnp.float32)],
    )(hidden_states, W, emb, b2, onehot)
    return out[0, 0]
